# baseline (speedup 1.0000x reference)
import jax
import jax.numpy as jnp
from jax.experimental import pallas as pl


def _identity_kernel(x_ref, o_ref):
    o_ref[...] = x_ref[...]


def _sage_conv(x, src, dst, Wl, bl, Wr):
    agg = jax.ops.segment_sum(x[src], dst, num_segments=x.shape[0])
    deg = jax.ops.segment_sum(jnp.ones((dst.shape[0],), x.dtype), dst, num_segments=x.shape[0])
    mean = agg / jnp.clip(deg, 1.0)[:, None]
    return mean @ Wl + bl + x @ Wr


def kernel(x, edge_index, batch, Wl1, bl1, Wr1, Wl2, bl2, Wr2, W1, b1, W2, b2):
    src, dst = edge_index[0], edge_index[1]
    h = jax.nn.relu(_sage_conv(x, src, dst, Wl1, bl1, Wr1))
    h = jax.nn.relu(_sage_conv(h, src, dst, Wl2, bl2, Wr2))
    nn = W1.shape[0] // h.shape[1]
    ng = batch.shape[0] // nn
    h = h.reshape(ng, -1)
    h = jax.nn.relu(h @ W1 + b1)
    out = h @ W2 + b2
    out = pl.pallas_call(
        _identity_kernel,
        out_shape=jax.ShapeDtypeStruct(out.shape, out.dtype),
    )(out)
    return out.astype(jnp.float32)


# speedup vs baseline: 2.1207x; 2.1207x over previous
import functools

import jax
import jax.numpy as jnp
from jax import lax
from jax.experimental import pallas as pl
from jax.experimental.pallas import tpu as pltpu
from jax.experimental.pallas import tpu_sc as plsc

NSUB = 16
NCORE = 2
CH = 128
HW = 128
NPASS = 4


def _sc_agg_builder(N, QN, QR, NCH, compute_deg):
    stripe = QR // NSUB

    def out_stripes(s, src_sh, dst_ref, base, qn_p):
        ns_full = qn_p // stripe
        rem = qn_p - ns_full * stripe
        @pl.when(s < ns_full)
        def _():
            pltpu.sync_copy(src_sh.at[pl.ds(s * stripe, stripe)],
                            dst_ref.at[pl.ds(base + s * stripe, stripe)])
        if rem > 0:
            @pl.when(s == ns_full)
            def _():
                pltpu.sync_copy(src_sh.at[pl.ds(ns_full * stripe, rem)],
                                dst_ref.at[pl.ds(base + ns_full * stripe, rem)])

    def body(xs_hbm, srcs_hbm, dstq_hbm, z_hbm, ones_hbm, *rest):
        if compute_deg:
            (agg_out, deg_out, src_v, dst_v, rows0, rows1, ones_v,
             acc_sh, deg_sh, sem0, sem1) = rest
        else:
            (agg_out, src_v, dst_v, rows0, rows1, ones_v,
             acc_sh, sem0, sem1) = rest
        c = lax.axis_index("c")
        s = lax.axis_index("s")

        pltpu.sync_copy(srcs_hbm.at[c, s], src_v)
        if compute_deg:
            pltpu.sync_copy(ones_hbm, ones_v)

        for p in range(NPASS):
            pltpu.sync_copy(dstq_hbm.at[p, s], dst_v)
            pltpu.sync_copy(z_hbm, acc_sh.at[pl.ds(s * stripe, stripe)])
            if compute_deg:
                @pl.when(c == 0)
                def _():
                    pltpu.sync_copy(z_hbm, deg_sh.at[pl.ds(s * stripe, stripe)])
            plsc.subcore_barrier()

            pltpu.async_copy(xs_hbm.at[src_v.at[0]], rows0, sem0)

            def scatter(jj, rows):
                pltpu.sync_copy(rows, acc_sh.at[dst_v.at[jj]], add=True)
                if compute_deg:
                    @pl.when(c == 0)
                    def _():
                        pltpu.sync_copy(ones_v, deg_sh.at[dst_v.at[jj]],
                                        add=True)

            def step(j2, carry):
                j = 2 * j2
                pltpu.async_copy(xs_hbm.at[src_v.at[j + 1]], rows1, sem1)
                pltpu.make_async_copy(xs_hbm.at[src_v.at[j]], rows0,
                                      sem0).wait()
                scatter(j, rows0)

                @pl.when(j2 < (NCH // 2) - 1)
                def _():
                    pltpu.async_copy(xs_hbm.at[src_v.at[j + 2]], rows0, sem0)
                pltpu.make_async_copy(xs_hbm.at[src_v.at[j + 1]], rows1,
                                      sem1).wait()
                scatter(j + 1, rows1)
                return carry

            lax.fori_loop(0, NCH // 2, step, 0)
            plsc.subcore_barrier()
            qn_p = min(QN, N - p * QN)
            out_stripes(s, acc_sh, agg_out.at[c], p * QN, qn_p)
            if compute_deg:
                @pl.when(c == 0)
                def _():
                    out_stripes(s, deg_sh, deg_out, p * QN, qn_p)
            plsc.subcore_barrier()

    out_type = [jax.ShapeDtypeStruct((NCORE, N, HW), jnp.float32)]
    scratch = [
        pltpu.VMEM((NCH, CH), jnp.int32),
        pltpu.VMEM((NCH, CH), jnp.int32),
        pltpu.VMEM((CH, HW), jnp.float32),
        pltpu.VMEM((CH, HW), jnp.float32),
        pltpu.VMEM((CH, HW), jnp.float32),
        pltpu.VMEM_SHARED((QR, HW), jnp.float32),
    ]
    if compute_deg:
        out_type.append(jax.ShapeDtypeStruct((N, HW), jnp.float32))
        scratch.append(pltpu.VMEM_SHARED((QR, HW), jnp.float32))
    scratch += [pltpu.SemaphoreType.DMA, pltpu.SemaphoreType.DMA]
    return pl.kernel(
        body,
        out_type=out_type,
        mesh=plsc.VectorSubcoreMesh(core_axis_name="c", subcore_axis_name="s"),
        scratch_types=scratch,
    )


_DOT = functools.partial(lax.dot_general, precision=jax.lax.Precision.HIGHEST,
                         preferred_element_type=jnp.float32)
_MM = lambda a, b: _DOT(a, b, (((1,), (0,)), ((), ())))


def _split_body(x_ref, out_ref):
    for q in range(NCORE):
        out_ref[q] = x_ref[:, q * HW:(q + 1) * HW]


def _split_cols(x, bm=2000):
    N, D = x.shape
    return pl.pallas_call(
        _split_body,
        grid=(N // bm,),
        in_specs=[pl.BlockSpec((bm, D), lambda i: (i, 0))],
        out_specs=pl.BlockSpec((NCORE, bm, HW), lambda i: (0, i, 0)),
        out_shape=jax.ShapeDtypeStruct((NCORE, N, HW), jnp.float32),
    )(x)


def _dense_layer_body(split_out, agg_ref, deg_ref, x_ref, wl_ref, bl_ref,
                      wr_ref, out_ref):
    invd = 1.0 / jnp.maximum(deg_ref[...], 1.0)
    h = bl_ref[...]
    for q in range(NCORE):
        h = h + _MM(agg_ref[q] * invd, wl_ref[q * HW:(q + 1) * HW, :])
        h = h + _MM(x_ref[q], wr_ref[q * HW:(q + 1) * HW, :])
    h = jnp.maximum(h, 0.0)
    if split_out:
        for q in range(NCORE):
            out_ref[q] = h[:, q * HW:(q + 1) * HW]
    else:
        out_ref[...] = h


def _dense_layer(agg, deg, xs, Wl, bl, Wr, split_out, bm=1000):
    N = deg.shape[0]
    D = Wl.shape[0]
    grid = (N // bm,)
    out_shape = (jax.ShapeDtypeStruct((NCORE, N, HW), jnp.float32) if split_out
                 else jax.ShapeDtypeStruct((N, D), jnp.float32))
    out_spec = (pl.BlockSpec((NCORE, bm, HW), lambda i: (0, i, 0)) if split_out
                else pl.BlockSpec((bm, D), lambda i: (i, 0)))
    return pl.pallas_call(
        functools.partial(_dense_layer_body, split_out),
        grid=grid,
        in_specs=[
            pl.BlockSpec((NCORE, bm, HW), lambda i: (0, i, 0)),
            pl.BlockSpec((bm, 1), lambda i: (i, 0)),
            pl.BlockSpec((NCORE, bm, HW), lambda i: (0, i, 0)),
            pl.BlockSpec((D, D), lambda i: (0, 0)),
            pl.BlockSpec((1, D), lambda i: (0, 0)),
            pl.BlockSpec((D, D), lambda i: (0, 0)),
        ],
        out_specs=out_spec,
        out_shape=out_shape,
    )(agg, deg, xs, Wl, bl, Wr)


def _head_body(nk, h_ref, w1_ref, b1_ref, w2_ref, b2_ref, out_ref, acc_ref):
    k = pl.program_id(0)

    @pl.when(k == 0)
    def _():
        acc_ref[...] = jnp.zeros_like(acc_ref)

    acc_ref[...] += _MM(h_ref[...], w1_ref[...])

    @pl.when(k == nk - 1)
    def _():
        h = jnp.maximum(acc_ref[...] + b1_ref[...], 0.0)
        out_ref[...] = _MM(h, w2_ref[...]) + b2_ref[...]


def _head(hflat, W1, b1, W2, b2, kb=512):
    NG = hflat.shape[0]
    K = W1.shape[0]
    MLP = W1.shape[1]
    OUT = W2.shape[1]
    nk = K // kb
    return pl.pallas_call(
        functools.partial(_head_body, nk),
        grid=(nk,),
        in_specs=[
            pl.BlockSpec((NG, kb), lambda k: (0, k)),
            pl.BlockSpec((kb, MLP), lambda k: (k, 0)),
            pl.BlockSpec((1, MLP), lambda k: (0, 0)),
            pl.BlockSpec((MLP, OUT), lambda k: (0, 0)),
            pl.BlockSpec((1, OUT), lambda k: (0, 0)),
        ],
        out_specs=pl.BlockSpec((NG, OUT), lambda k: (0, 0)),
        out_shape=jax.ShapeDtypeStruct((NG, OUT), jnp.float32),
        scratch_shapes=[pltpu.VMEM((NG, MLP), jnp.float32)],
    )(hflat, W1, b1, W2, b2)


def kernel(x, edge_index, batch, Wl1, bl1, Wr1, Wl2, bl2, Wr2, W1, b1, W2, b2):
    N, D = x.shape
    E = edge_index.shape[1]

    QN = -(-N // (NPASS * 128)) * 128
    QR = QN + 128
    trash = 128
    stripe = QR // NSUB

    NCH = -(-E // (NSUB * CH))
    if NCH % 2:
        NCH += 1
    EP = NSUB * NCH * CH
    pad = EP - E

    src = edge_index[0]
    dst = edge_index[1]
    pad_ar = jnp.arange(pad, dtype=jnp.int32)
    src_p = jnp.concatenate([src, pad_ar % N]).reshape(NSUB, NCH, CH)
    dst_p = jnp.concatenate([dst, jnp.full((pad,), N, jnp.int32)])
    qs = []
    for p in range(NPASS):
        lo = p * QN
        inq = (dst_p >= lo) & (dst_p < lo + QN)
        qs.append(jnp.where(inq, dst_p - lo, QN + dst_p % trash))
    dst_q = jnp.stack(qs).reshape(NPASS, NSUB, NCH, CH)
    coff = (jnp.arange(NCORE, dtype=jnp.int32) * N)[:, None, None, None]
    srcs_b = src_p[None] + coff

    z = jnp.zeros((stripe, HW), jnp.float32)
    ones_c = jnp.ones((CH, HW), jnp.float32)

    xs = _split_cols(x)

    agg_k1 = _sc_agg_builder(N, QN, QR, NCH, compute_deg=True)
    agg_k2 = _sc_agg_builder(N, QN, QR, NCH, compute_deg=False)

    agg1, degw = agg_k1(xs.reshape(NCORE * N, HW), srcs_b, dst_q, z, ones_c)
    deg = degw[:, :1]

    h1s = _dense_layer(agg1, deg, xs, Wl1, bl1.reshape(1, -1), Wr1,
                       split_out=True)

    (agg2,) = agg_k2(h1s.reshape(NCORE * N, HW), srcs_b, dst_q, z, ones_c)

    h2 = _dense_layer(agg2, deg, h1s, Wl2, bl2.reshape(1, -1), Wr2,
                      split_out=False)

    nn = W1.shape[0] // h2.shape[1]
    ng = batch.shape[0] // nn
    hflat = h2.reshape(ng, nn * h2.shape[1])
    out = _head(hflat, W1, b1.reshape(1, -1), W2, b2.reshape(1, -1))
    return out.astype(jnp.float32)


# speedup vs baseline: 2.1691x; 1.0228x over previous
import functools

import jax
import jax.numpy as jnp
from jax import lax
from jax.experimental import pallas as pl
from jax.experimental.pallas import tpu as pltpu
from jax.experimental.pallas import tpu_sc as plsc

NSUB = 16
NCORE = 2
CH = 128
HW = 128
NPASS = 4


def _sc_agg_builder(N, NR, QN, QR, NCH, npass, compute_deg):
    stripe = QR // NSUB

    def out_stripes(s, src_sh, dst_ref, base, qn_p, st):
        ns_full = qn_p // st
        rem = qn_p - ns_full * st
        @pl.when(s < ns_full)
        def _():
            pltpu.sync_copy(src_sh.at[pl.ds(s * st, st)],
                            dst_ref.at[pl.ds(base + s * st, st)])
        if rem > 0:
            @pl.when(s == ns_full)
            def _():
                pltpu.sync_copy(src_sh.at[pl.ds(ns_full * st, rem)],
                                dst_ref.at[pl.ds(base + ns_full * st, rem)])

    def body(xs_hbm, srcs_hbm, dstq_hbm, z_hbm, ones_hbm, *rest):
        if compute_deg:
            (agg_out, deg_out, src_v, dst_v, rows0, rows1, ones_v,
             acc_sh, deg_sh, sem0, sem1) = rest
        else:
            (agg_out, src_v, dst_v, rows0, rows1,
             acc_sh, sem0, sem1) = rest
        c = lax.axis_index("c")
        s = lax.axis_index("s")

        pltpu.sync_copy(srcs_hbm.at[c, s], src_v)
        if compute_deg:
            pltpu.sync_copy(ones_hbm, ones_v)

        for p in range(npass):
            pltpu.sync_copy(dstq_hbm.at[p, s], dst_v)
            pltpu.sync_copy(z_hbm, acc_sh.at[pl.ds(s * stripe, stripe)])
            if compute_deg:
                @pl.when(c == 0)
                def _():
                    pltpu.sync_copy(z_hbm, deg_sh.at[pl.ds(s * stripe,
                                                           stripe)])
            plsc.subcore_barrier()

            pltpu.async_copy(xs_hbm.at[src_v.at[0]], rows0, sem0)

            def scatter(jj, rows):
                pltpu.sync_copy(rows, acc_sh.at[dst_v.at[jj]], add=True)
                if compute_deg:
                    @pl.when(c == 0)
                    def _():
                        pltpu.sync_copy(ones_v, deg_sh.at[dst_v.at[jj]],
                                        add=True)

            def step(j2, carry):
                j = 2 * j2
                pltpu.async_copy(xs_hbm.at[src_v.at[j + 1]], rows1, sem1)
                pltpu.make_async_copy(xs_hbm.at[src_v.at[j]], rows0,
                                      sem0).wait()
                scatter(j, rows0)

                @pl.when(j2 < (NCH // 2) - 1)
                def _():
                    pltpu.async_copy(xs_hbm.at[src_v.at[j + 2]], rows0, sem0)
                pltpu.make_async_copy(xs_hbm.at[src_v.at[j + 1]], rows1,
                                      sem1).wait()
                scatter(j + 1, rows1)
                return carry

            lax.fori_loop(0, NCH // 2, step, 0)
            plsc.subcore_barrier()
            qn_p = min(QN, N - p * QN)
            out_stripes(s, acc_sh, agg_out.at[c], p * QN, qn_p, stripe)
            if compute_deg:
                @pl.when(c == 0)
                def _():
                    out_stripes(s, deg_sh, deg_out, p * QN, qn_p, stripe)
            plsc.subcore_barrier()

    out_type = [jax.ShapeDtypeStruct((NCORE, N, HW), jnp.float32)]
    scratch = [
        pltpu.VMEM((NCH, CH), jnp.int32),
        pltpu.VMEM((NCH, CH), jnp.int32),
        pltpu.VMEM((CH, HW), jnp.float32),
        pltpu.VMEM((CH, HW), jnp.float32),
    ]
    if compute_deg:
        scratch.append(pltpu.VMEM((CH, HW), jnp.float32))
    scratch.append(pltpu.VMEM_SHARED((QR, HW), jnp.float32))
    if compute_deg:
        out_type.append(jax.ShapeDtypeStruct((N, HW), jnp.float32))
        scratch.append(pltpu.VMEM_SHARED((QR, HW), jnp.float32))
    scratch += [pltpu.SemaphoreType.DMA, pltpu.SemaphoreType.DMA]
    return pl.kernel(
        body,
        out_type=out_type,
        mesh=plsc.VectorSubcoreMesh(core_axis_name="c", subcore_axis_name="s"),
        scratch_types=scratch,
    )


_DOT = functools.partial(lax.dot_general, precision=jax.lax.Precision.HIGHEST,
                         preferred_element_type=jnp.float32)
_MM = lambda a, b: _DOT(a, b, (((1,), (0,)), ((), ())))


def _prep_edges_body(N, E, QNa, QNb, npa, npb, ei_ref, srcs_ref, dstqa_ref,
                     dstqb_ref, srcp_ref, dstp_ref):
    EPT = srcs_ref.shape[1]
    SPT = EPT // NSUB
    EPS = E // NSUB
    padn = SPT - EPS
    for s in range(NSUB):
        srcp_ref[:, s * SPT:s * SPT + EPS] = ei_ref[0:1, s * EPS:(s + 1) * EPS]
        dstp_ref[:, s * SPT:s * SPT + EPS] = ei_ref[1:2, s * EPS:(s + 1) * EPS]
        iot = lax.broadcasted_iota(jnp.int32, (1, padn), 1) + s * padn
        srcp_ref[:, s * SPT + EPS:(s + 1) * SPT] = lax.rem(iot, N)
        dstp_ref[:, s * SPT + EPS:(s + 1) * SPT] = jnp.full((1, padn), N,
                                                            jnp.int32)
    srcp = srcp_ref[...]
    dstp = dstp_ref[...]
    for c in range(NCORE):
        srcs_ref[c] = srcp[0] + c * N
    for p in range(npa):
        lo = p * QNa
        inq = (dstp >= lo) & (dstp < lo + QNa)
        dstqa_ref[p] = jnp.where(inq, dstp - lo, QNa + lax.rem(dstp, 128))[0]
    for p in range(npb):
        lo = p * QNb
        inq = (dstp >= lo) & (dstp < lo + QNb)
        dstqb_ref[p] = jnp.where(inq, dstp - lo, QNb + lax.rem(dstp, 128))[0]


def _prep_edges(edge_index, N, QNa, QNb, npa, npb, NCH):
    E = edge_index.shape[1]
    EPT = NSUB * NCH * CH
    kb = functools.partial(_prep_edges_body, N, E, QNa, QNb, npa, npb)
    return pl.pallas_call(
        kb,
        grid=(1,),
        in_specs=[pl.BlockSpec((2, E), lambda i: (0, 0))],
        out_specs=[
            pl.BlockSpec((NCORE, EPT), lambda i: (0, 0)),
            pl.BlockSpec((npa, EPT), lambda i: (0, 0)),
            pl.BlockSpec((npb, EPT), lambda i: (0, 0)),
        ],
        out_shape=[
            jax.ShapeDtypeStruct((NCORE, EPT), jnp.int32),
            jax.ShapeDtypeStruct((npa, EPT), jnp.int32),
            jax.ShapeDtypeStruct((npb, EPT), jnp.int32),
        ],
        scratch_shapes=[pltpu.VMEM((1, EPT), jnp.int32),
                        pltpu.VMEM((1, EPT), jnp.int32)],
    )(edge_index)


def _split_body(x_ref, out_ref):
    for q in range(NCORE):
        out_ref[q] = x_ref[:, q * HW:(q + 1) * HW]


def _split_cols(x, bm=2000):
    N, D = x.shape
    return pl.pallas_call(
        _split_body,
        grid=(N // bm,),
        in_specs=[pl.BlockSpec((bm, D), lambda i: (i, 0))],
        out_specs=pl.BlockSpec((NCORE, bm, HW), lambda i: (0, i, 0)),
        out_shape=jax.ShapeDtypeStruct((NCORE, N, HW), jnp.float32),
    )(x)


def _dense_layer_body(split_out, agg_ref, deg_ref, x_ref, wl_ref, bl_ref,
                      wr_ref, out_ref):
    invd = 1.0 / jnp.maximum(deg_ref[...], 1.0)
    h = bl_ref[...]
    for q in range(NCORE):
        h = h + _MM(agg_ref[q] * invd, wl_ref[q * HW:(q + 1) * HW, :])
        h = h + _MM(x_ref[q], wr_ref[q * HW:(q + 1) * HW, :])
    h = jnp.maximum(h, 0.0)
    if split_out:
        for q in range(NCORE):
            out_ref[q] = h[:, q * HW:(q + 1) * HW]
    else:
        out_ref[...] = h


def _dense_layer(agg, deg, xs, Wl, bl, Wr, split_out, bm=1000):
    N = deg.shape[0]
    D = Wl.shape[0]
    grid = (N // bm,)
    out_shape = (jax.ShapeDtypeStruct((NCORE, N, HW), jnp.float32) if split_out
                 else jax.ShapeDtypeStruct((N, D), jnp.float32))
    out_spec = (pl.BlockSpec((NCORE, bm, HW), lambda i: (0, i, 0)) if split_out
                else pl.BlockSpec((bm, D), lambda i: (i, 0)))
    return pl.pallas_call(
        functools.partial(_dense_layer_body, split_out),
        grid=grid,
        in_specs=[
            pl.BlockSpec((NCORE, bm, HW), lambda i: (0, i, 0)),
            pl.BlockSpec((bm, 1), lambda i: (i, 0)),
            pl.BlockSpec((NCORE, bm, HW), lambda i: (0, i, 0)),
            pl.BlockSpec((D, D), lambda i: (0, 0)),
            pl.BlockSpec((1, D), lambda i: (0, 0)),
            pl.BlockSpec((D, D), lambda i: (0, 0)),
        ],
        out_specs=out_spec,
        out_shape=out_shape,
    )(agg, deg, xs, Wl, bl, Wr)


def _head_body(nk, h_ref, w1_ref, b1_ref, w2_ref, b2_ref, out_ref, acc_ref):
    k = pl.program_id(0)

    @pl.when(k == 0)
    def _():
        acc_ref[...] = jnp.zeros_like(acc_ref)

    acc_ref[...] += _MM(h_ref[...], w1_ref[...])

    @pl.when(k == nk - 1)
    def _():
        h = jnp.maximum(acc_ref[...] + b1_ref[...], 0.0)
        out_ref[...] = _MM(h, w2_ref[...]) + b2_ref[...]


def _head(hflat, W1, b1, W2, b2, kb=512):
    NG = hflat.shape[0]
    K = W1.shape[0]
    MLP = W1.shape[1]
    OUT = W2.shape[1]
    nk = K // kb
    return pl.pallas_call(
        functools.partial(_head_body, nk),
        grid=(nk,),
        in_specs=[
            pl.BlockSpec((NG, kb), lambda k: (0, k)),
            pl.BlockSpec((kb, MLP), lambda k: (k, 0)),
            pl.BlockSpec((1, MLP), lambda k: (0, 0)),
            pl.BlockSpec((MLP, OUT), lambda k: (0, 0)),
            pl.BlockSpec((1, OUT), lambda k: (0, 0)),
        ],
        out_specs=pl.BlockSpec((NG, OUT), lambda k: (0, 0)),
        out_shape=jax.ShapeDtypeStruct((NG, OUT), jnp.float32),
        scratch_shapes=[pltpu.VMEM((NG, MLP), jnp.float32)],
    )(hflat, W1, b1, W2, b2)


def kernel(x, edge_index, batch, Wl1, bl1, Wr1, Wl2, bl2, Wr2, W1, b1, W2, b2):
    N, D = x.shape
    E = edge_index.shape[1]

    NCH = -(-E // (NSUB * CH))
    if NCH % 2:
        NCH += 1
    EP = NSUB * NCH * CH
    pad = EP - E

    NPASS1, NPASS2 = 4, 3
    QN1 = -(-N // (NPASS1 * 128)) * 128
    QR1 = QN1 + 128
    QN2 = -(-N // (NPASS2 * 128)) * 128
    QR2 = QN2 + 128
    z1 = jnp.zeros((QR1 // NSUB, HW), jnp.float32)
    z2 = jnp.zeros((QR2 // NSUB, HW), jnp.float32)
    ones_c = jnp.ones((CH, HW), jnp.float32)

    srcs2, dstqa, dstqb = _prep_edges(edge_index, N, QN1, QN2, NPASS1,
                                      NPASS2, NCH)
    srcs_b = srcs2.reshape(NCORE, NSUB, NCH, CH)
    dst_q1 = dstqa.reshape(NPASS1, NSUB, NCH, CH)
    dst_q2 = dstqb.reshape(NPASS2, NSUB, NCH, CH)

    xs = _split_cols(x)

    agg_k1 = _sc_agg_builder(N, 0, QN1, QR1, NCH, NPASS1, compute_deg=True)
    agg_k2 = _sc_agg_builder(N, 0, QN2, QR2, NCH, NPASS2, compute_deg=False)

    agg1, degw = agg_k1(xs.reshape(NCORE * N, HW), srcs_b, dst_q1, z1,
                        ones_c)
    deg = degw[:, :1]

    h1s = _dense_layer(agg1, deg, xs, Wl1, bl1.reshape(1, -1), Wr1,
                       split_out=True)

    (agg2,) = agg_k2(h1s.reshape(NCORE * N, HW), srcs_b, dst_q2, z2, ones_c)

    h2 = _dense_layer(agg2, deg, h1s, Wl2, bl2.reshape(1, -1), Wr2,
                      split_out=False)

    nn = W1.shape[0] // h2.shape[1]
    ng = batch.shape[0] // nn
    hflat = h2.reshape(ng, nn * h2.shape[1])
    out = _head(hflat, W1, b1.reshape(1, -1), W2, b2.reshape(1, -1))
    return out.astype(jnp.float32)


# speedup vs baseline: 2.8001x; 1.2909x over previous
import functools

import jax
import jax.numpy as jnp
from jax import lax
from jax.experimental import pallas as pl
from jax.experimental.pallas import tpu as pltpu
from jax.experimental.pallas import tpu_sc as plsc

NSUB = 16
NCORE = 2
CH = 128
HW = 128
NPASS = 4


def _sc_agg_builder(N, NR, QN, QR, NCH, npass, compute_deg):
    stripe = QR // NSUB

    def out_stripes(s, src_sh, dst_ref, base, qn_p, st):
        ns_full = qn_p // st
        rem = qn_p - ns_full * st
        @pl.when(s < ns_full)
        def _():
            pltpu.sync_copy(src_sh.at[pl.ds(s * st, st)],
                            dst_ref.at[pl.ds(base + s * st, st)])
        if rem > 0:
            @pl.when(s == ns_full)
            def _():
                pltpu.sync_copy(src_sh.at[pl.ds(ns_full * st, rem)],
                                dst_ref.at[pl.ds(base + ns_full * st, rem)])

    def body(xs_hbm, srcs_hbm, dstq_hbm, z_hbm, ones_hbm, *rest):
        if compute_deg:
            (agg_out, deg_out, src_v, dst_v, rows0, rows1, ones_v,
             acc_sh, deg_sh, sem0, sem1) = rest
        else:
            (agg_out, src_v, dst_v, rows0, rows1,
             acc_sh, sem0, sem1) = rest
        c = lax.axis_index("c")
        s = lax.axis_index("s")

        pltpu.sync_copy(srcs_hbm.at[c, s], src_v)
        if compute_deg:
            pltpu.sync_copy(ones_hbm, ones_v)

        for p in range(npass):
            dcore = p % NCORE
            pltpu.sync_copy(dstq_hbm.at[p, s], dst_v)
            pltpu.sync_copy(z_hbm, acc_sh.at[pl.ds(s * stripe, stripe)])
            if compute_deg:
                @pl.when(c == dcore)
                def _():
                    pltpu.sync_copy(z_hbm, deg_sh.at[pl.ds(s * stripe,
                                                           stripe)])
            plsc.subcore_barrier()

            pltpu.async_copy(xs_hbm.at[src_v.at[0]], rows0, sem0)

            def scatter(jj, rows):
                pltpu.sync_copy(rows, acc_sh.at[dst_v.at[jj]], add=True)
                if compute_deg:
                    @pl.when(c == dcore)
                    def _():
                        pltpu.sync_copy(ones_v, deg_sh.at[dst_v.at[jj]],
                                        add=True)

            def step(j2, carry):
                j = 2 * j2
                pltpu.async_copy(xs_hbm.at[src_v.at[j + 1]], rows1, sem1)
                pltpu.make_async_copy(xs_hbm.at[src_v.at[j]], rows0,
                                      sem0).wait()
                scatter(j, rows0)

                @pl.when(j2 < (NCH // 2) - 1)
                def _():
                    pltpu.async_copy(xs_hbm.at[src_v.at[j + 2]], rows0, sem0)
                pltpu.make_async_copy(xs_hbm.at[src_v.at[j + 1]], rows1,
                                      sem1).wait()
                scatter(j + 1, rows1)
                return carry

            lax.fori_loop(0, NCH // 2, step, 0)
            plsc.subcore_barrier()
            qn_p = min(QN, N - p * QN)
            out_stripes(s, acc_sh, agg_out.at[c], p * QN, qn_p, stripe)
            if compute_deg:
                @pl.when(c == dcore)
                def _():
                    out_stripes(s, deg_sh, deg_out, p * QN, qn_p, stripe)
            plsc.subcore_barrier()

    out_type = [jax.ShapeDtypeStruct((NCORE, N, HW), jnp.float32)]
    scratch = [
        pltpu.VMEM((NCH, CH), jnp.int32),
        pltpu.VMEM((NCH, CH), jnp.int32),
        pltpu.VMEM((CH, HW), jnp.float32),
        pltpu.VMEM((CH, HW), jnp.float32),
    ]
    if compute_deg:
        scratch.append(pltpu.VMEM((CH, HW), jnp.float32))
    scratch.append(pltpu.VMEM_SHARED((QR, HW), jnp.float32))
    if compute_deg:
        out_type.append(jax.ShapeDtypeStruct((N, HW), jnp.float32))
        scratch.append(pltpu.VMEM_SHARED((QR, HW), jnp.float32))
    scratch += [pltpu.SemaphoreType.DMA, pltpu.SemaphoreType.DMA]
    return pl.kernel(
        body,
        out_type=out_type,
        mesh=plsc.VectorSubcoreMesh(core_axis_name="c", subcore_axis_name="s"),
        scratch_types=scratch,
    )


_DOT = functools.partial(lax.dot_general, precision=jax.lax.Precision.HIGHEST,
                         preferred_element_type=jnp.float32)
_MM = lambda a, b: _DOT(a, b, (((1,), (0,)), ((), ())))


def _prep_edges_body(N, E, QNa, QNb, npa, npb, ei_ref, srcs_ref, dstqa_ref,
                     dstqb_ref, srcp_ref, dstp_ref):
    EPT = srcs_ref.shape[1]
    SPT = EPT // NSUB
    EPS = E // NSUB
    padn = SPT - EPS
    for s in range(NSUB):
        srcp_ref[:, s * SPT:s * SPT + EPS] = ei_ref[0:1, s * EPS:(s + 1) * EPS]
        dstp_ref[:, s * SPT:s * SPT + EPS] = ei_ref[1:2, s * EPS:(s + 1) * EPS]
        iot = lax.broadcasted_iota(jnp.int32, (1, padn), 1) + s * padn
        srcp_ref[:, s * SPT + EPS:(s + 1) * SPT] = lax.rem(iot, N)
        dstp_ref[:, s * SPT + EPS:(s + 1) * SPT] = jnp.full((1, padn), N,
                                                            jnp.int32)
    srcp = srcp_ref[...]
    dstp = dstp_ref[...]
    for c in range(NCORE):
        srcs_ref[c] = srcp[0] + c * N
    for p in range(npa):
        lo = p * QNa
        inq = (dstp >= lo) & (dstp < lo + QNa)
        dstqa_ref[p] = jnp.where(inq, dstp - lo, QNa + lax.rem(dstp, 128))[0]
    for p in range(npb):
        lo = p * QNb
        inq = (dstp >= lo) & (dstp < lo + QNb)
        dstqb_ref[p] = jnp.where(inq, dstp - lo, QNb + lax.rem(dstp, 128))[0]


def _prep_edges(edge_index, N, QNa, QNb, npa, npb, NCH):
    E = edge_index.shape[1]
    EPT = NSUB * NCH * CH
    kb = functools.partial(_prep_edges_body, N, E, QNa, QNb, npa, npb)
    return pl.pallas_call(
        kb,
        grid=(1,),
        in_specs=[pl.BlockSpec((2, E), lambda i: (0, 0))],
        out_specs=[
            pl.BlockSpec((NCORE, EPT), lambda i: (0, 0)),
            pl.BlockSpec((npa, EPT), lambda i: (0, 0)),
            pl.BlockSpec((npb, EPT), lambda i: (0, 0)),
        ],
        out_shape=[
            jax.ShapeDtypeStruct((NCORE, EPT), jnp.int32),
            jax.ShapeDtypeStruct((npa, EPT), jnp.int32),
            jax.ShapeDtypeStruct((npb, EPT), jnp.int32),
        ],
        scratch_shapes=[pltpu.VMEM((1, EPT), jnp.int32),
                        pltpu.VMEM((1, EPT), jnp.int32)],
    )(edge_index)


def _split_body(x_ref, out_ref):
    for q in range(NCORE):
        out_ref[q] = x_ref[:, q * HW:(q + 1) * HW]


def _split_cols(x, bm=2000):
    N, D = x.shape
    return pl.pallas_call(
        _split_body,
        grid=(N // bm,),
        in_specs=[pl.BlockSpec((bm, D), lambda i: (i, 0))],
        out_specs=pl.BlockSpec((NCORE, bm, HW), lambda i: (0, i, 0)),
        out_shape=jax.ShapeDtypeStruct((NCORE, N, HW), jnp.float32),
    )(x)


def _dense_layer_body(split_out, agg_ref, deg_ref, x_ref, wl_ref, bl_ref,
                      wr_ref, out_ref):
    invd = 1.0 / jnp.maximum(deg_ref[...], 1.0)
    h = bl_ref[...]
    for q in range(NCORE):
        h = h + _MM(agg_ref[q] * invd, wl_ref[q * HW:(q + 1) * HW, :])
        h = h + _MM(x_ref[q], wr_ref[q * HW:(q + 1) * HW, :])
    h = jnp.maximum(h, 0.0)
    if split_out:
        for q in range(NCORE):
            out_ref[q] = h[:, q * HW:(q + 1) * HW]
    else:
        out_ref[...] = h


def _dense_layer(agg, deg, xs, Wl, bl, Wr, split_out, bm=1000):
    N = deg.shape[0]
    D = Wl.shape[0]
    grid = (N // bm,)
    out_shape = (jax.ShapeDtypeStruct((NCORE, N, HW), jnp.float32) if split_out
                 else jax.ShapeDtypeStruct((N, D), jnp.float32))
    out_spec = (pl.BlockSpec((NCORE, bm, HW), lambda i: (0, i, 0)) if split_out
                else pl.BlockSpec((bm, D), lambda i: (i, 0)))
    return pl.pallas_call(
        functools.partial(_dense_layer_body, split_out),
        grid=grid,
        in_specs=[
            pl.BlockSpec((NCORE, bm, HW), lambda i: (0, i, 0)),
            pl.BlockSpec((bm, 1), lambda i: (i, 0)),
            pl.BlockSpec((NCORE, bm, HW), lambda i: (0, i, 0)),
            pl.BlockSpec((D, D), lambda i: (0, 0)),
            pl.BlockSpec((1, D), lambda i: (0, 0)),
            pl.BlockSpec((D, D), lambda i: (0, 0)),
        ],
        out_specs=out_spec,
        out_shape=out_shape,
    )(agg, deg, xs, Wl, bl, Wr)


def _head_body(nk, h_ref, w1_ref, b1_ref, w2_ref, b2_ref, out_ref, acc_ref):
    k = pl.program_id(0)

    @pl.when(k == 0)
    def _():
        acc_ref[...] = jnp.zeros_like(acc_ref)

    acc_ref[...] += _MM(h_ref[...], w1_ref[...])

    @pl.when(k == nk - 1)
    def _():
        h = jnp.maximum(acc_ref[...] + b1_ref[...], 0.0)
        out_ref[...] = _MM(h, w2_ref[...]) + b2_ref[...]


def _head(hflat, W1, b1, W2, b2, kb=512):
    NG = hflat.shape[0]
    K = W1.shape[0]
    MLP = W1.shape[1]
    OUT = W2.shape[1]
    nk = K // kb
    return pl.pallas_call(
        functools.partial(_head_body, nk),
        grid=(nk,),
        in_specs=[
            pl.BlockSpec((NG, kb), lambda k: (0, k)),
            pl.BlockSpec((kb, MLP), lambda k: (k, 0)),
            pl.BlockSpec((1, MLP), lambda k: (0, 0)),
            pl.BlockSpec((MLP, OUT), lambda k: (0, 0)),
            pl.BlockSpec((1, OUT), lambda k: (0, 0)),
        ],
        out_specs=pl.BlockSpec((NG, OUT), lambda k: (0, 0)),
        out_shape=jax.ShapeDtypeStruct((NG, OUT), jnp.float32),
        scratch_shapes=[pltpu.VMEM((NG, MLP), jnp.float32)],
    )(hflat, W1, b1, W2, b2)


def kernel(x, edge_index, batch, Wl1, bl1, Wr1, Wl2, bl2, Wr2, W1, b1, W2, b2):
    N, D = x.shape
    E = edge_index.shape[1]

    NCH = -(-E // (NSUB * CH))
    if NCH % 2:
        NCH += 1
    EP = NSUB * NCH * CH
    pad = EP - E

    NPASS1, NPASS2 = 3, 3
    QN1 = -(-N // (NPASS1 * 128)) * 128
    QR1 = QN1 + 128
    QN2 = -(-N // (NPASS2 * 128)) * 128
    QR2 = QN2 + 128
    z1 = jnp.zeros((QR1 // NSUB, HW), jnp.float32)
    z2 = jnp.zeros((QR2 // NSUB, HW), jnp.float32)
    ones_c = jnp.ones((CH, HW), jnp.float32)

    srcs2, dstqa, dstqb = _prep_edges(edge_index, N, QN1, QN2, NPASS1,
                                      NPASS2, NCH)
    srcs_b = srcs2.reshape(NCORE, NSUB, NCH, CH)
    dst_q1 = dstqa.reshape(NPASS1, NSUB, NCH, CH)
    dst_q2 = dstqb.reshape(NPASS2, NSUB, NCH, CH)

    xs = _split_cols(x)

    agg_k1 = _sc_agg_builder(N, 0, QN1, QR1, NCH, NPASS1, compute_deg=True)
    agg_k2 = _sc_agg_builder(N, 0, QN2, QR2, NCH, NPASS2, compute_deg=False)

    agg1, degw = agg_k1(xs.reshape(NCORE * N, HW), srcs_b, dst_q1, z1,
                        ones_c)
    deg = degw[:, :1]

    h1s = _dense_layer(agg1, deg, xs, Wl1, bl1.reshape(1, -1), Wr1,
                       split_out=True)

    (agg2,) = agg_k2(h1s.reshape(NCORE * N, HW), srcs_b, dst_q2, z2, ones_c)

    h2 = _dense_layer(agg2, deg, h1s, Wl2, bl2.reshape(1, -1), Wr2,
                      split_out=False)

    nn = W1.shape[0] // h2.shape[1]
    ng = batch.shape[0] // nn
    hflat = h2.reshape(ng, nn * h2.shape[1])
    out = _head(hflat, W1, b1.reshape(1, -1), W2, b2.reshape(1, -1))
    return out.astype(jnp.float32)


# speedup vs baseline: 3.1192x; 1.1140x over previous
import functools

import jax
import jax.numpy as jnp
from jax import lax
from jax.experimental import pallas as pl
from jax.experimental.pallas import tpu as pltpu
from jax.experimental.pallas import tpu_sc as plsc

NSUB = 16
NCORE = 2
CH = 128
HW = 128
NPASS = 4


def _sc_agg_builder(N, NR, QN, QR, NCH, npass, compute_deg):
    stripe = QR // NSUB

    def out_stripes(s, src_sh, dst_ref, base, qn_p, st):
        ns_full = qn_p // st
        rem = qn_p - ns_full * st
        @pl.when(s < ns_full)
        def _():
            pltpu.sync_copy(src_sh.at[pl.ds(s * st, st)],
                            dst_ref.at[pl.ds(base + s * st, st)])
        if rem > 0:
            @pl.when(s == ns_full)
            def _():
                pltpu.sync_copy(src_sh.at[pl.ds(ns_full * st, rem)],
                                dst_ref.at[pl.ds(base + ns_full * st, rem)])

    def body(xs_hbm, srcs_hbm, dstq_hbm, z_hbm, ones_hbm, *rest):
        if compute_deg:
            (agg_out, deg_out, src_v, dst_v, rows0, rows1, ones_v,
             acc_sh, deg_sh, sem0, sem1) = rest
        else:
            (agg_out, src_v, dst_v, rows0, rows1,
             acc_sh, sem0, sem1) = rest
        c = lax.axis_index("c")
        s = lax.axis_index("s")

        pltpu.sync_copy(srcs_hbm.at[c, s], src_v)
        if compute_deg:
            pltpu.sync_copy(ones_hbm, ones_v)

        for p in range(npass):
            dcore = p % NCORE
            pltpu.sync_copy(dstq_hbm.at[p, s], dst_v)
            pltpu.sync_copy(z_hbm, acc_sh.at[pl.ds(s * stripe, stripe)])
            if compute_deg:
                @pl.when(c == dcore)
                def _():
                    pltpu.sync_copy(z_hbm, deg_sh.at[pl.ds(s * stripe,
                                                           stripe)])
            plsc.subcore_barrier()

            pltpu.async_copy(xs_hbm.at[src_v.at[0]], rows0, sem0)

            def scatter(jj, rows):
                pltpu.sync_copy(rows, acc_sh.at[dst_v.at[jj]], add=True)
                if compute_deg:
                    @pl.when(c == dcore)
                    def _():
                        pltpu.sync_copy(ones_v, deg_sh.at[dst_v.at[jj]],
                                        add=True)

            def step(j2, carry):
                j = 2 * j2
                pltpu.async_copy(xs_hbm.at[src_v.at[j + 1]], rows1, sem1)
                pltpu.make_async_copy(xs_hbm.at[src_v.at[j]], rows0,
                                      sem0).wait()
                scatter(j, rows0)

                @pl.when(j2 < (NCH // 2) - 1)
                def _():
                    pltpu.async_copy(xs_hbm.at[src_v.at[j + 2]], rows0, sem0)
                pltpu.make_async_copy(xs_hbm.at[src_v.at[j + 1]], rows1,
                                      sem1).wait()
                scatter(j + 1, rows1)
                return carry

            lax.fori_loop(0, NCH // 2, step, 0)
            plsc.subcore_barrier()
            qn_p = min(QN, N - p * QN)
            out_stripes(s, acc_sh, agg_out.at[c], p * QN, qn_p, stripe)
            if compute_deg:
                @pl.when(c == dcore)
                def _():
                    out_stripes(s, deg_sh, deg_out, p * QN, qn_p, stripe)
            plsc.subcore_barrier()

    out_type = [jax.ShapeDtypeStruct((NCORE, N, HW), jnp.float32)]
    scratch = [
        pltpu.VMEM((NCH, CH), jnp.int32),
        pltpu.VMEM((NCH, CH), jnp.int32),
        pltpu.VMEM((CH, HW), jnp.float32),
        pltpu.VMEM((CH, HW), jnp.float32),
    ]
    if compute_deg:
        scratch.append(pltpu.VMEM((CH, HW), jnp.float32))
    scratch.append(pltpu.VMEM_SHARED((QR, HW), jnp.float32))
    if compute_deg:
        out_type.append(jax.ShapeDtypeStruct((N, HW), jnp.float32))
        scratch.append(pltpu.VMEM_SHARED((QR, HW), jnp.float32))
    scratch += [pltpu.SemaphoreType.DMA, pltpu.SemaphoreType.DMA]
    return pl.kernel(
        body,
        out_type=out_type,
        mesh=plsc.VectorSubcoreMesh(core_axis_name="c", subcore_axis_name="s"),
        scratch_types=scratch,
    )


_DOT = functools.partial(lax.dot_general, precision=jax.lax.Precision.HIGHEST,
                         preferred_element_type=jnp.float32)
_MM = lambda a, b: _DOT(a, b, (((1,), (0,)), ((), ())))


def _prep_edges_body(N, E, QNa, QNb, npa, npb, ei_ref, srcs_ref, dstqa_ref,
                     dstqb_ref, srcp_ref, dstp_ref):
    EPT = srcs_ref.shape[1]
    SPT = EPT // NSUB
    EPS = E // NSUB
    padn = SPT - EPS
    for s in range(NSUB):
        srcp_ref[:, s * SPT:s * SPT + EPS] = ei_ref[0:1, s * EPS:(s + 1) * EPS]
        dstp_ref[:, s * SPT:s * SPT + EPS] = ei_ref[1:2, s * EPS:(s + 1) * EPS]
        iot = lax.broadcasted_iota(jnp.int32, (1, padn), 1) + s * padn
        srcp_ref[:, s * SPT + EPS:(s + 1) * SPT] = lax.rem(iot, N)
        dstp_ref[:, s * SPT + EPS:(s + 1) * SPT] = jnp.full((1, padn), N,
                                                            jnp.int32)
    srcp = srcp_ref[...]
    dstp = dstp_ref[...]
    for c in range(NCORE):
        srcs_ref[c] = srcp[0] + c * N
    for p in range(npa):
        lo = p * QNa
        inq = (dstp >= lo) & (dstp < lo + QNa)
        dstqa_ref[p] = jnp.where(inq, dstp - lo, QNa + lax.rem(dstp, 128))[0]
    for p in range(npb):
        lo = p * QNb
        inq = (dstp >= lo) & (dstp < lo + QNb)
        dstqb_ref[p] = jnp.where(inq, dstp - lo, QNb + lax.rem(dstp, 128))[0]


def _prep_edges(edge_index, N, QNa, QNb, npa, npb, NCH):
    E = edge_index.shape[1]
    EPT = NSUB * NCH * CH
    kb = functools.partial(_prep_edges_body, N, E, QNa, QNb, npa, npb)
    return pl.pallas_call(
        kb,
        grid=(1,),
        in_specs=[pl.BlockSpec((2, E), lambda i: (0, 0))],
        out_specs=[
            pl.BlockSpec((NCORE, EPT), lambda i: (0, 0)),
            pl.BlockSpec((npa, EPT), lambda i: (0, 0)),
            pl.BlockSpec((npb, EPT), lambda i: (0, 0)),
        ],
        out_shape=[
            jax.ShapeDtypeStruct((NCORE, EPT), jnp.int32),
            jax.ShapeDtypeStruct((npa, EPT), jnp.int32),
            jax.ShapeDtypeStruct((npb, EPT), jnp.int32),
        ],
        scratch_shapes=[pltpu.VMEM((1, EPT), jnp.int32),
                        pltpu.VMEM((1, EPT), jnp.int32)],
    )(edge_index)


def _split_body(x_ref, out_ref):
    for q in range(NCORE):
        out_ref[q] = x_ref[:, q * HW:(q + 1) * HW]


def _split_cols(x, bm=2000):
    N, D = x.shape
    return pl.pallas_call(
        _split_body,
        grid=(N // bm,),
        in_specs=[pl.BlockSpec((bm, D), lambda i: (i, 0))],
        out_specs=pl.BlockSpec((NCORE, bm, HW), lambda i: (0, i, 0)),
        out_shape=jax.ShapeDtypeStruct((NCORE, N, HW), jnp.float32),
    )(x)


def _dense_layer_body(split_out, agg_ref, deg_ref, x_ref, wl_ref, bl_ref,
                      wr_ref, out_ref):
    invd = 1.0 / jnp.maximum(deg_ref[...], 1.0)
    h = bl_ref[...]
    for q in range(NCORE):
        h = h + _MM(agg_ref[q] * invd, wl_ref[q * HW:(q + 1) * HW, :])
        h = h + _MM(x_ref[q], wr_ref[q * HW:(q + 1) * HW, :])
    h = jnp.maximum(h, 0.0)
    if split_out:
        for q in range(NCORE):
            out_ref[q] = h[:, q * HW:(q + 1) * HW]
    else:
        out_ref[...] = h


def _dense_layer(agg, deg, xs, Wl, bl, Wr, split_out, bm=1000):
    N = deg.shape[0]
    D = Wl.shape[0]
    grid = (N // bm,)
    out_shape = (jax.ShapeDtypeStruct((NCORE, N, HW), jnp.float32) if split_out
                 else jax.ShapeDtypeStruct((N, D), jnp.float32))
    out_spec = (pl.BlockSpec((NCORE, bm, HW), lambda i: (0, i, 0)) if split_out
                else pl.BlockSpec((bm, D), lambda i: (i, 0)))
    return pl.pallas_call(
        functools.partial(_dense_layer_body, split_out),
        grid=grid,
        in_specs=[
            pl.BlockSpec((NCORE, bm, HW), lambda i: (0, i, 0)),
            pl.BlockSpec((bm, 1), lambda i: (i, 0)),
            pl.BlockSpec((NCORE, bm, HW), lambda i: (0, i, 0)),
            pl.BlockSpec((D, D), lambda i: (0, 0)),
            pl.BlockSpec((1, D), lambda i: (0, 0)),
            pl.BlockSpec((D, D), lambda i: (0, 0)),
        ],
        out_specs=out_spec,
        out_shape=out_shape,
    )(agg, deg, xs, Wl, bl, Wr)


def _head_body(nk, h_ref, w1_ref, b1_ref, w2_ref, b2_ref, out_ref, acc_ref):
    k = pl.program_id(0)

    @pl.when(k == 0)
    def _():
        acc_ref[...] = jnp.zeros_like(acc_ref)

    acc_ref[...] += _MM(h_ref[...], w1_ref[...])

    @pl.when(k == nk - 1)
    def _():
        h = jnp.maximum(acc_ref[...] + b1_ref[...], 0.0)
        out_ref[...] = _MM(h, w2_ref[...]) + b2_ref[...]


def _head(hflat, W1, b1, W2, b2, kb=512):
    NG = hflat.shape[0]
    K = W1.shape[0]
    MLP = W1.shape[1]
    OUT = W2.shape[1]
    nk = K // kb
    return pl.pallas_call(
        functools.partial(_head_body, nk),
        grid=(nk,),
        in_specs=[
            pl.BlockSpec((NG, kb), lambda k: (0, k)),
            pl.BlockSpec((kb, MLP), lambda k: (k, 0)),
            pl.BlockSpec((1, MLP), lambda k: (0, 0)),
            pl.BlockSpec((MLP, OUT), lambda k: (0, 0)),
            pl.BlockSpec((1, OUT), lambda k: (0, 0)),
        ],
        out_specs=pl.BlockSpec((NG, OUT), lambda k: (0, 0)),
        out_shape=jax.ShapeDtypeStruct((NG, OUT), jnp.float32),
        scratch_shapes=[pltpu.VMEM((NG, MLP), jnp.float32)],
    )(hflat, W1, b1, W2, b2)


def kernel(x, edge_index, batch, Wl1, bl1, Wr1, Wl2, bl2, Wr2, W1, b1, W2, b2):
    N, D = x.shape
    E = edge_index.shape[1]

    NCH = -(-E // (NSUB * CH))
    if NCH % 2:
        NCH += 1
    EP = NSUB * NCH * CH
    pad = EP - E

    NPASS1, NPASS2 = 3, 2
    QN1 = -(-N // (NPASS1 * 128)) * 128
    QR1 = QN1 + 128
    QN2 = -(-N // (NPASS2 * 128)) * 128
    QR2 = QN2 + 128
    z1 = jnp.zeros((QR1 // NSUB, HW), jnp.float32)
    z2 = jnp.zeros((QR2 // NSUB, HW), jnp.float32)
    ones_c = jnp.ones((CH, HW), jnp.float32)

    srcs2, dstqa, dstqb = _prep_edges(edge_index, N, QN1, QN2, NPASS1,
                                      NPASS2, NCH)
    srcs_b = srcs2.reshape(NCORE, NSUB, NCH, CH)
    dst_q1 = dstqa.reshape(NPASS1, NSUB, NCH, CH)
    dst_q2 = dstqb.reshape(NPASS2, NSUB, NCH, CH)

    xs = _split_cols(x)

    agg_k1 = _sc_agg_builder(N, 0, QN1, QR1, NCH, NPASS1, compute_deg=True)
    agg_k2 = _sc_agg_builder(N, 0, QN2, QR2, NCH, NPASS2, compute_deg=False)

    agg1, degw = agg_k1(xs.reshape(NCORE * N, HW), srcs_b, dst_q1, z1,
                        ones_c)
    deg = degw[:, :1]

    h1s = _dense_layer(agg1, deg, xs, Wl1, bl1.reshape(1, -1), Wr1,
                       split_out=True)

    (agg2,) = agg_k2(h1s.reshape(NCORE * N, HW), srcs_b, dst_q2, z2, ones_c)

    h2 = _dense_layer(agg2, deg, h1s, Wl2, bl2.reshape(1, -1), Wr2,
                      split_out=False)

    nn = W1.shape[0] // h2.shape[1]
    ng = batch.shape[0] // nn
    hflat = h2.reshape(ng, nn * h2.shape[1])
    out = _head(hflat, W1, b1.reshape(1, -1), W2, b2.reshape(1, -1))
    return out.astype(jnp.float32)
